# Initial kernel scaffold; baseline (speedup 1.0000x reference)
import jax
import jax.numpy as jnp
from jax import lax
from jax.experimental import pallas as pl
from jax.experimental.pallas import tpu as pltpu
from jax.experimental.pallas import tpu_sc as plsc

VOCAB = 1_000_000
D = 64
B_TOTAL = 16384 * 50
CHUNK = 128
NC, NS = 2, 16
NW = NC * NS
BPW = B_TOTAL // NW
NCH = BPW // CHUNK


def _body(idx_hbm, table_hbm, out_hbm, idx_v, buf_v, gsem):
    wid = lax.axis_index("s") * NC + lax.axis_index("c")
    pltpu.sync_copy(idx_hbm.at[pl.ds(wid * NCH, NCH)], idx_v)
    base = wid * BPW

    @pl.loop(0, NCH)
    def _chunk(j):
        pltpu.async_copy(table_hbm.at[idx_v.at[j]], buf_v, gsem).wait()
        pltpu.sync_copy(buf_v, out_hbm.at[pl.ds(base + j * CHUNK, CHUNK)])


@jax.jit
def _embed(idx2d, weight):
    mesh = plsc.VectorSubcoreMesh(core_axis_name="c", subcore_axis_name="s")
    return pl.kernel(
        _body,
        out_type=jax.ShapeDtypeStruct((B_TOTAL, D), jnp.float32),
        mesh=mesh,
        scratch_types=[
            pltpu.VMEM((NCH, CHUNK), jnp.int32),
            pltpu.VMEM((CHUNK, D), jnp.float32),
            pltpu.SemaphoreType.DMA,
        ],
    )(idx2d, weight)


def kernel(token_ids, weight):
    idx2d = token_ids.reshape(-1).astype(jnp.int32).reshape(NW * NCH, CHUNK)
    out = _embed(idx2d, weight)
    return out.reshape(token_ids.shape[0], token_ids.shape[1], D)


# speedup vs baseline: 1.6849x; 1.6849x over previous
import jax
import jax.numpy as jnp
from jax import lax
from jax.experimental import pallas as pl
from jax.experimental.pallas import tpu as pltpu
from jax.experimental.pallas import tpu_sc as plsc

VOCAB = 1_000_000
D = 64
B_TOTAL = 16384 * 50
CHUNK = 128
NC, NS = 2, 16
NW = NC * NS
BPW = B_TOTAL // NW
NCH = BPW // CHUNK


def _body(idx_hbm, table_hbm, out_hbm, idx_v, buf_v, gsem):
    wid = lax.axis_index("s") * NC + lax.axis_index("c")
    pltpu.sync_copy(idx_hbm.at[pl.ds(wid * NCH, NCH)], idx_v)
    base = wid * BPW

    @pl.loop(0, NCH)
    def _chunk(j):
        pltpu.async_copy(table_hbm.at[idx_v.at[j]], buf_v, gsem).wait()
        pltpu.sync_copy(buf_v, out_hbm.at[pl.ds(base + j * CHUNK, CHUNK)])


@jax.jit
def _embed(idx2d, weight):
    mesh = plsc.VectorSubcoreMesh(core_axis_name="c", subcore_axis_name="s")
    return pl.kernel(
        _body,
        out_type=jax.ShapeDtypeStruct((B_TOTAL, D), jnp.float32),
        mesh=mesh,
        scratch_types=[
            pltpu.VMEM((NCH, CHUNK), jnp.int32),
            pltpu.VMEM((CHUNK, D), jnp.float32),
            pltpu.SemaphoreType.DMA,
        ],
        compiler_params=pltpu.CompilerParams(use_tc_tiling_on_sc=False),
    )(idx2d, weight)


def kernel(token_ids, weight):
    idx2d = token_ids.reshape(-1).astype(jnp.int32).reshape(NW * NCH, CHUNK)
    out = _embed(idx2d, weight)
    return out.reshape(token_ids.shape[0], token_ids.shape[1], D)


# speedup vs baseline: 1.8750x; 1.1128x over previous
import jax
import jax.numpy as jnp
from jax import lax
from jax.experimental import pallas as pl
from jax.experimental.pallas import tpu as pltpu
from jax.experimental.pallas import tpu_sc as plsc

VOCAB = 1_000_000
D = 64
B_TOTAL = 16384 * 50
CHUNK = 128
NC, NS = 2, 16
NW = NC * NS
BPW = B_TOTAL // NW
NCH = BPW // CHUNK
NBUF = 8
T = NCH // NBUF


def _body(idx_hbm, table_hbm, out_hbm, idx_v, buf_v, gsem, ssem):
    wid = lax.axis_index("s") * NC + lax.axis_index("c")
    pltpu.sync_copy(idx_hbm.at[pl.ds(wid * NCH, NCH)], idx_v)
    base = wid * BPW

    def fire_gathers(t, drain_prev):
        descs = []
        for b in range(NBUF):
            j = t * NBUF + b
            if drain_prev:
                pltpu.make_async_copy(
                    buf_v.at[b], out_hbm.at[pl.ds(base, CHUNK)],
                    ssem.at[b]).wait()
            descs.append(
                pltpu.async_copy(table_hbm.at[idx_v.at[j]], buf_v.at[b],
                                 gsem.at[b]))
        return descs

    def drain_and_scatter(t, gdescs):
        for b in range(NBUF):
            j = t * NBUF + b
            gdescs[b].wait()
            pltpu.async_copy(
                buf_v.at[b], out_hbm.at[pl.ds(base + j * CHUNK, CHUNK)],
                ssem.at[b])

    gdescs = fire_gathers(0, drain_prev=False)
    drain_and_scatter(0, gdescs)

    @pl.loop(1, T)
    def _cycle(t):
        gd = fire_gathers(t, drain_prev=True)
        drain_and_scatter(t, gd)

    for b in range(NBUF):
        pltpu.make_async_copy(
            buf_v.at[b], out_hbm.at[pl.ds(base, CHUNK)], ssem.at[b]).wait()


@jax.jit
def _embed(idx2d, weight):
    mesh = plsc.VectorSubcoreMesh(core_axis_name="c", subcore_axis_name="s")
    return pl.kernel(
        _body,
        out_type=jax.ShapeDtypeStruct((B_TOTAL, D), jnp.float32),
        mesh=mesh,
        scratch_types=[
            pltpu.VMEM((NCH, CHUNK), jnp.int32),
            pltpu.VMEM((NBUF, CHUNK, D), jnp.float32),
            pltpu.SemaphoreType.DMA((NBUF,)),
            pltpu.SemaphoreType.DMA((NBUF,)),
        ],
        compiler_params=pltpu.CompilerParams(use_tc_tiling_on_sc=False),
    )(idx2d, weight)


def kernel(token_ids, weight):
    idx2d = token_ids.reshape(-1).astype(jnp.int32).reshape(NW * NCH, CHUNK)
    out = _embed(idx2d, weight)
    return out.reshape(token_ids.shape[0], token_ids.shape[1], D)
